# baseline (speedup 1.0000x reference)
import functools

import jax
import jax.numpy as jnp
from jax import lax
from jax.experimental import pallas as pl
from jax.experimental.pallas import tpu as pltpu
from jax.experimental.pallas import tpu_sc as plsc

_NUM_PATCHES = 576
_PROJ_DIM = 768
_NUM_CORES = 2
_NUM_SUBCORES = 16
_ROWS_PER_WORKER = 24
_ACTIVE_WORKERS = _NUM_PATCHES // _ROWS_PER_WORKER


@functools.partial(
    pl.kernel,
    mesh=plsc.VectorSubcoreMesh(core_axis_name="c", subcore_axis_name="s"),
    out_type=jax.ShapeDtypeStruct((_NUM_PATCHES, _PROJ_DIM), jnp.float32),
)
def _sc_lookup(table_hbm, out_hbm):
    wid = lax.axis_index("s") * _NUM_CORES + lax.axis_index("c")

    @pl.when(wid < _ACTIVE_WORKERS)
    def _():
        base = pl.multiple_of(wid * _ROWS_PER_WORKER, 8)
        pltpu.sync_copy(
            table_hbm.at[pl.ds(base, _ROWS_PER_WORKER)],
            out_hbm.at[pl.ds(base, _ROWS_PER_WORKER)],
        )


def kernel(patch, pos_table):
    del patch
    return _sc_lookup(pos_table)


# speedup vs baseline: 1.0220x; 1.0220x over previous
import functools

import jax
import jax.numpy as jnp
from jax import lax
from jax.experimental import pallas as pl
from jax.experimental.pallas import tpu as pltpu
from jax.experimental.pallas import tpu_sc as plsc

_NUM_PATCHES = 576
_PROJ_DIM = 768
_NUM_CORES = 2
_NUM_SUBCORES = 16
_HALF = _NUM_PATCHES // 2


@functools.partial(
    pl.kernel,
    mesh=plsc.ScalarSubcoreMesh(axis_name="c", num_cores=_NUM_CORES),
    out_type=jax.ShapeDtypeStruct((_NUM_PATCHES, _PROJ_DIM), jnp.float32),
)
def _sc_lookup(table_hbm, out_hbm):
    cid = lax.axis_index("c")
    base = pl.multiple_of(cid * _HALF, 8)
    pltpu.sync_copy(
        table_hbm.at[pl.ds(base, _HALF)],
        out_hbm.at[pl.ds(base, _HALF)],
    )


def kernel(patch, pos_table):
    del patch
    return _sc_lookup(pos_table)


# speedup vs baseline: 28.8939x; 28.2731x over previous
import functools

import jax
import jax.numpy as jnp
from jax import lax
from jax.experimental import pallas as pl
from jax.experimental.pallas import tpu as pltpu
from jax.experimental.pallas import tpu_sc as plsc

_NUM_PATCHES = 576
_PROJ_DIM = 768
_NUM_CORES = 2
_NUM_SUBCORES = 16
_CHUNKS = ((0, 192), (192, 192), (384, 176), (560, 16))


def _overlap_body(table_hbm, out_hbm, buf, in_sems, out_sems):
    for k, (base, rows) in enumerate(_CHUNKS):
        pltpu.make_async_copy(
            table_hbm.at[pl.ds(base, rows)],
            buf.at[pl.ds(base, rows)],
            in_sems.at[k],
        ).start()
    for k, (base, rows) in enumerate(_CHUNKS):
        pltpu.make_async_copy(
            table_hbm.at[pl.ds(base, rows)],
            buf.at[pl.ds(base, rows)],
            in_sems.at[k],
        ).wait()
        pltpu.make_async_copy(
            buf.at[pl.ds(base, rows)],
            out_hbm.at[pl.ds(base, rows)],
            out_sems.at[k],
        ).start()
    for k, (base, rows) in enumerate(_CHUNKS):
        pltpu.make_async_copy(
            buf.at[pl.ds(base, rows)],
            out_hbm.at[pl.ds(base, rows)],
            out_sems.at[k],
        ).wait()


def kernel(patch, pos_table):
    del patch
    return pl.pallas_call(
        _overlap_body,
        in_specs=[pl.BlockSpec(memory_space=pl.ANY)],
        out_specs=pl.BlockSpec(memory_space=pl.ANY),
        out_shape=jax.ShapeDtypeStruct((_NUM_PATCHES, _PROJ_DIM), jnp.float32),
        scratch_shapes=[
            pltpu.VMEM((_NUM_PATCHES, _PROJ_DIM), jnp.float32),
            pltpu.SemaphoreType.DMA((len(_CHUNKS),)),
            pltpu.SemaphoreType.DMA((len(_CHUNKS),)),
        ],
    )(pos_table)
